# Initial kernel scaffold; baseline (speedup 1.0000x reference)
import jax
import jax.numpy as jnp
from jax.experimental import pallas as pl


def kernel(x, edge_index, edge_weight, ori_edge_index, ori_edge_weight, W):
    raise NotImplementedError("write your pallas kernel here")


# speedup vs baseline: 1.0495x; 1.0495x over previous
import functools

import jax
import jax.numpy as jnp
from jax.experimental import pallas as pl

_N = 10000
_E = 320000
_D = 128


def _matmul_body(x_ref, w_ref, o_ref):
    o_ref[...] = jnp.dot(x_ref[...], w_ref[...],
                         preferred_element_type=jnp.float32)


def _matmul(x, w):
    blk = 2000
    return pl.pallas_call(
        _matmul_body,
        grid=(_N // blk,),
        in_specs=[
            pl.BlockSpec((blk, _D), lambda i: (i, 0)),
            pl.BlockSpec((_D, _D), lambda i: (0, 0)),
        ],
        out_specs=pl.BlockSpec((blk, _D), lambda i: (i, 0)),
        out_shape=jax.ShapeDtypeStruct((_N, _D), jnp.float32),
    )(x, w)


def kernel(x, edge_index, edge_weight, ori_edge_index, ori_edge_weight, W):
    pre_sup = _matmul(x, W)

    src, dst = edge_index[0], edge_index[1]
    msgs = pre_sup[src] * edge_weight[:, None]
    out1 = jax.ops.segment_sum(msgs, dst, num_segments=_N)

    ori_src, ori_dst = ori_edge_index[0], ori_edge_index[1]
    ori_msgs = pre_sup[ori_src] * ori_edge_weight[:, None]
    out2 = jax.ops.segment_sum(ori_msgs, ori_dst, num_segments=_N)

    return jax.nn.relu(out1), jax.nn.relu(out2)


# speedup vs baseline: 3.4871x; 3.3225x over previous
import jax
import jax.numpy as jnp
from jax import lax
from jax.experimental import pallas as pl
from jax.experimental.pallas import tpu as pltpu
from jax.experimental.pallas import tpu_sc as plsc

_N = 10000
_E = 320000
_D = 128

_NTILES = 16
_K = 80
_EPT = _E // _NTILES
_NCHUNK = _EPT // _K
_WBTILES = 10
_RPT = _N // _WBTILES
_ZROWS = 40


def _matmul_body(x_ref, w_ref, o_ref):
    o_ref[...] = jnp.dot(x_ref[...], w_ref[...],
                         preferred_element_type=jnp.float32)


def _matmul(x, w):
    blk = 2000
    return pl.pallas_call(
        _matmul_body,
        grid=(_N // blk,),
        in_specs=[
            pl.BlockSpec((blk, _D), lambda i: (i, 0)),
            pl.BlockSpec((_D, _D), lambda i: (0, 0)),
        ],
        out_specs=pl.BlockSpec((blk, _D), lambda i: (i, 0)),
        out_shape=jax.ShapeDtypeStruct((_N, _D), jnp.float32),
    )(x, w)


def _gcn_body(pre_hbm, src_hbm, dst_hbm, w_hbm, out_hbm,
              acc, rows_v, zbuf, src_v, dst_v, w_v, sem):
    c = lax.axis_index("c")
    s = lax.axis_index("s")

    zeros = jnp.zeros((16,), jnp.float32)

    def _zrow(r, carry):
        for j in range(8):
            zbuf[r, pl.ds(j * 16, 16)] = zeros
        return carry
    lax.fori_loop(0, _ZROWS, _zrow, 0)

    @pl.when(s < _WBTILES)
    def _():
        def _zcp(i, carry):
            pltpu.sync_copy(zbuf, acc.at[pl.ds(s * _RPT + i * _ZROWS, _ZROWS)])
            return carry
        lax.fori_loop(0, _RPT // _ZROWS, _zcp, 0)

    plsc.subcore_barrier()

    ebase = c * _E + s * _EPT

    def _chunk(i, carry):
        base = ebase + i * _K
        pltpu.sync_copy(src_hbm.at[pl.ds(base, _K)], src_v)
        pltpu.sync_copy(dst_hbm.at[pl.ds(base, _K)], dst_v)
        pltpu.sync_copy(w_hbm.at[pl.ds(base, _K)], w_v)
        pltpu.async_copy(pre_hbm.at[src_v], rows_v, sem).wait()

        def _scale16(k16, carry2):
            wv = w_v[pl.ds(k16 * 16, 16)]
            for e in range(16):
                wk = wv[e]
                k = k16 * 16 + e
                for j in range(8):
                    sl = (k, pl.ds(j * 16, 16))
                    rows_v[sl] = rows_v[sl] * wk
            return carry2
        lax.fori_loop(0, _K // 16, _scale16, 0)

        pltpu.sync_copy(rows_v, acc.at[dst_v], add=True)
        return carry
    lax.fori_loop(0, _NCHUNK, _chunk, 0)

    plsc.subcore_barrier()

    @pl.when(s < _WBTILES)
    def _():
        def _wb(i, carry):
            rb = s * _RPT + i * _ZROWS
            pltpu.sync_copy(acc.at[pl.ds(rb, _ZROWS)],
                            rows_v.at[pl.ds(0, _ZROWS)])

            def _relu_row(r, carry2):
                for j in range(8):
                    sl = (r, pl.ds(j * 16, 16))
                    rows_v[sl] = jnp.maximum(rows_v[sl], 0.0)
                return carry2
            lax.fori_loop(0, _ZROWS, _relu_row, 0)

            pltpu.sync_copy(rows_v.at[pl.ds(0, _ZROWS)],
                            out_hbm.at[pl.ds(c * _N + rb, _ZROWS)])
            return carry
        lax.fori_loop(0, _RPT // _ZROWS, _wb, 0)


def _edge_pass(pre_sup, src, dst, w):
    mesh = plsc.VectorSubcoreMesh(core_axis_name="c", subcore_axis_name="s")
    return pl.kernel(
        _gcn_body,
        out_type=jax.ShapeDtypeStruct((2 * _N, _D), jnp.float32),
        mesh=mesh,
        scratch_types=[
            pltpu.VMEM_SHARED((_N, _D), jnp.float32),
            pltpu.VMEM((_K, _D), jnp.float32),
            pltpu.VMEM((_ZROWS, _D), jnp.float32),
            pltpu.VMEM((_K,), jnp.int32),
            pltpu.VMEM((_K,), jnp.int32),
            pltpu.VMEM((_K,), jnp.float32),
            pltpu.SemaphoreType.DMA,
        ],
    )(pre_sup, src, dst, w)


def kernel(x, edge_index, edge_weight, ori_edge_index, ori_edge_weight, W):
    pre_sup = _matmul(x, W)
    src = jnp.concatenate([edge_index[0], ori_edge_index[0]]).astype(jnp.int32)
    dst = jnp.concatenate([edge_index[1], ori_edge_index[1]]).astype(jnp.int32)
    w = jnp.concatenate([edge_weight, ori_edge_weight])
    out = _edge_pass(pre_sup, src, dst, w)
    return out[:_N], out[_N:]
